# baseline (speedup 1.0000x reference)
import functools

import jax
import jax.numpy as jnp
from jax import lax
from jax.experimental import pallas as pl
from jax.experimental.pallas import tpu as pltpu
from jax.experimental.pallas import tpu_sc as plsc

_NC = 2
_NS = 16
_NW = _NC * _NS
_CH = 32
_NBUF = 3


@functools.lru_cache(maxsize=None)
def _make_lookup(BA, SEQ, V, D):
    B = BA * SEQ
    assert B % (_NW * _CH) == 0, (BA, SEQ)
    bpw = B // _NW
    nch = bpw // _CH
    assert SEQ % bpw == 0
    wpr = SEQ // bpw

    mesh = plsc.VectorSubcoreMesh(core_axis_name="c", subcore_axis_name="s")

    @functools.partial(
        pl.kernel,
        mesh=mesh,
        out_type=jax.ShapeDtypeStruct((BA, SEQ, D), jnp.float32),
        scratch_types=[
            pltpu.VMEM((bpw,), jnp.int32),
        ] + [pltpu.VMEM((_CH, D), jnp.float32) for _ in range(_NBUF)] + [
            pltpu.SemaphoreType.DMA,
            pltpu.SemaphoreType.DMA,
        ],
    )
    def lookup(ids_hbm, table_hbm, out_hbm, idx_v, *rest):
        bufs = rest[:_NBUF]
        gsem, ssem = rest[_NBUF:]
        wid = lax.axis_index("s") * _NC + lax.axis_index("c")
        b = wid // wpr
        s0 = (wid % wpr) * bpw
        pltpu.sync_copy(ids_hbm.at[b, pl.ds(s0, bpw)], idx_v)

        gathers = [None] * _NBUF
        scatters = [None] * _NBUF
        for c in range(nch):
            s = c % _NBUF
            if scatters[s] is not None:
                scatters[s].wait()
            gathers[s] = pltpu.async_copy(
                table_hbm.at[idx_v.at[pl.ds(c * _CH, _CH)]], bufs[s], gsem)
            if c >= 1:
                p = (c - 1) % _NBUF
                gathers[p].wait()
                scatters[p] = pltpu.async_copy(
                    bufs[p], out_hbm.at[b, pl.ds(s0 + (c - 1) * _CH, _CH)],
                    ssem)
        last = (nch - 1) % _NBUF
        gathers[last].wait()
        scatters[last] = pltpu.async_copy(
            bufs[last], out_hbm.at[b, pl.ds(s0 + (nch - 1) * _CH, _CH)], ssem)
        for c in range(max(0, nch - _NBUF), nch):
            scatters[c % _NBUF].wait()

    return lookup


def kernel(input_ids, wte):
    in_shape = input_ids.shape
    ids = input_ids.reshape((-1, in_shape[-1]))
    if ids.dtype != jnp.int32:
        ids = ids.astype(jnp.int32)
    return _make_lookup(ids.shape[0], ids.shape[1],
                        wte.shape[0], wte.shape[1])(ids, wte)


# speedup vs baseline: 1.0017x; 1.0017x over previous
import functools

import jax
import jax.numpy as jnp
from jax import lax
from jax.experimental import pallas as pl
from jax.experimental.pallas import tpu as pltpu
from jax.experimental.pallas import tpu_sc as plsc

_NC = 2
_NS = 16
_NW = _NC * _NS
_CH = 16
_NBUF = 3


@functools.lru_cache(maxsize=None)
def _make_lookup(BA, SEQ, V, D):
    B = BA * SEQ
    assert B % (_NW * _CH) == 0, (BA, SEQ)
    bpw = B // _NW
    nch = bpw // _CH
    assert SEQ % bpw == 0
    wpr = SEQ // bpw

    mesh = plsc.VectorSubcoreMesh(core_axis_name="c", subcore_axis_name="s")

    @functools.partial(
        pl.kernel,
        mesh=mesh,
        out_type=jax.ShapeDtypeStruct((BA, SEQ, D), jnp.float32),
        scratch_types=[
            pltpu.VMEM((bpw,), jnp.int32),
        ] + [pltpu.VMEM((_CH, D), jnp.float32) for _ in range(_NBUF)] + [
            pltpu.SemaphoreType.DMA,
            pltpu.SemaphoreType.DMA,
        ],
    )
    def lookup(ids_hbm, table_hbm, out_hbm, idx_v, *rest):
        bufs = rest[:_NBUF]
        gsem, ssem = rest[_NBUF:]
        wid = lax.axis_index("s") * _NC + lax.axis_index("c")
        b = wid // wpr
        s0 = (wid % wpr) * bpw
        pltpu.sync_copy(ids_hbm.at[b, pl.ds(s0, bpw)], idx_v)

        gathers = [None] * _NBUF
        scatters = [None] * _NBUF
        for c in range(nch):
            s = c % _NBUF
            if scatters[s] is not None:
                scatters[s].wait()
            gathers[s] = pltpu.async_copy(
                table_hbm.at[idx_v.at[pl.ds(c * _CH, _CH)]], bufs[s], gsem)
            if c >= 1:
                p = (c - 1) % _NBUF
                gathers[p].wait()
                scatters[p] = pltpu.async_copy(
                    bufs[p], out_hbm.at[b, pl.ds(s0 + (c - 1) * _CH, _CH)],
                    ssem)
        last = (nch - 1) % _NBUF
        gathers[last].wait()
        scatters[last] = pltpu.async_copy(
            bufs[last], out_hbm.at[b, pl.ds(s0 + (nch - 1) * _CH, _CH)], ssem)
        for c in range(max(0, nch - _NBUF), nch):
            scatters[c % _NBUF].wait()

    return lookup


def kernel(input_ids, wte):
    in_shape = input_ids.shape
    ids = input_ids.reshape((-1, in_shape[-1]))
    if ids.dtype != jnp.int32:
        ids = ids.astype(jnp.int32)
    return _make_lookup(ids.shape[0], ids.shape[1],
                        wte.shape[0], wte.shape[1])(ids, wte)
